# baseline (speedup 1.0000x reference)
import functools

import jax
import jax.numpy as jnp
from jax import lax
from jax.experimental import pallas as pl
from jax.experimental.pallas import tpu as pltpu
from jax.experimental.pallas import tpu_sc as plsc

_NC = 2
_NS = 16
_CH = 128


def _make_deg_kernel(NP, CPT):
    DW = 16
    RPT = NP // _NS
    mesh = plsc.VectorSubcoreMesh(core_axis_name="c", subcore_axis_name="s")

    @functools.partial(
        pl.kernel,
        out_type=jax.ShapeDtypeStruct((_NC * NP, DW), jnp.float32),
        mesh=mesh,
        scratch_types=[
            pltpu.VMEM((CPT, _CH), jnp.int32),
            pltpu.VMEM((_CH, DW), jnp.float32),
            pltpu.VMEM((16, DW), jnp.float32),
            pltpu.VMEM_SHARED((NP, DW), jnp.float32),
        ],
    )
    def deg_kernel(dst_hbm, out_hbm, dst_v, ones_v, zbuf, acc):
        cid = lax.axis_index("c")
        sid = lax.axis_index("s")
        w = cid * _NS + sid
        for i in range(16):
            zbuf[i, :] = jnp.zeros((DW,), jnp.float32)
        for i in range(_CH):
            ones_v[i, :] = jnp.ones((DW,), jnp.float32)

        def zero_body(k, _):
            pltpu.sync_copy(zbuf, acc.at[pl.ds(sid * RPT + k * 16, 16)])
            return 0

        lax.fori_loop(0, RPT // 16, zero_body, 0)
        pltpu.sync_copy(dst_hbm.at[pl.ds(w * CPT, CPT)], dst_v)
        plsc.subcore_barrier()

        def body(j, _):
            pltpu.sync_copy(ones_v, acc.at[dst_v.at[j]], add=True)
            return 0

        lax.fori_loop(0, CPT, body, 0)
        plsc.subcore_barrier()
        pltpu.sync_copy(acc.at[pl.ds(sid * RPT, RPT)],
                        out_hbm.at[pl.ds(cid * NP + sid * RPT, RPT)])

    return deg_kernel


_SEG = 48


def _make_gs_kernel(NP, CPT0, CPT1, D):
    RPT = NP // _NS
    mesh = plsc.VectorSubcoreMesh(core_axis_name="c", subcore_axis_name="s")

    @functools.partial(
        pl.kernel,
        out_type=jax.ShapeDtypeStruct((_NC * NP, D), jnp.float32),
        mesh=mesh,
        scratch_types=[
            pltpu.VMEM((_SEG, _CH), jnp.int32),
            pltpu.VMEM((_SEG, _CH), jnp.int32),
            pltpu.VMEM((_CH, D), jnp.float32),
            pltpu.VMEM((_CH, D), jnp.float32),
            pltpu.VMEM((8, D), jnp.float32),
            pltpu.VMEM_SHARED((NP, D), jnp.float32),
            pltpu.SemaphoreType.DMA,
            pltpu.SemaphoreType.DMA,
        ],
    )
    def gs_kernel(y_hbm, src_hbm, dst_hbm, out_hbm,
                  src_v, dst_v, rows_a, rows_b, zbuf, acc, sem_a, sem_b):
        cid = lax.axis_index("c")
        sid = lax.axis_index("s")
        for i in range(8):
            for g in range(D // 16):
                zbuf[i, pl.ds(g * 16, 16)] = jnp.zeros((16,), jnp.float32)

        def zero_body(k, _):
            pltpu.sync_copy(zbuf, acc.at[pl.ds(sid * RPT + k * 8, 8)])
            return 0

        lax.fori_loop(0, RPT // 8, zero_body, 0)
        plsc.subcore_barrier()

        cpt = jnp.where(cid == 0, CPT0, CPT1)
        base = pl.multiple_of(
            jnp.where(cid == 0, sid * CPT0, _NS * CPT0 + sid * CPT1), 8)
        nseg = (cpt + _SEG - 1) // _SEG

        def seg_body(s, _):
            off = pl.multiple_of(base + s * _SEG, 8)
            pltpu.sync_copy(src_hbm.at[pl.ds(off, _SEG)], src_v)
            pltpu.sync_copy(dst_hbm.at[pl.ds(off, _SEG)], dst_v)
            cnt = jnp.minimum(cpt - s * _SEG, _SEG)
            pltpu.async_copy(y_hbm.at[src_v.at[0]], rows_a, sem_a)

            def pair(jj, _):
                j0 = 2 * jj
                j1 = j0 + 1
                pltpu.async_copy(y_hbm.at[src_v.at[j1]], rows_b, sem_b)
                pltpu.make_async_copy(y_hbm.at[src_v.at[j0]],
                                      rows_a, sem_a).wait()
                pltpu.sync_copy(rows_a, acc.at[dst_v.at[j0]], add=True)

                @pl.when(j0 + 2 < cnt)
                def _():
                    pltpu.async_copy(y_hbm.at[src_v.at[j0 + 2]],
                                     rows_a, sem_a)

                pltpu.make_async_copy(y_hbm.at[src_v.at[j1]],
                                      rows_b, sem_b).wait()
                pltpu.sync_copy(rows_b, acc.at[dst_v.at[j1]], add=True)
                return 0

            lax.fori_loop(0, cnt // 2, pair, 0)
            return 0

        lax.fori_loop(0, nseg, seg_body, 0)
        plsc.subcore_barrier()
        pltpu.sync_copy(acc.at[pl.ds(sid * RPT, RPT)],
                        out_hbm.at[pl.ds(cid * NP + sid * RPT, RPT)])

    return gs_kernel


def _dinv_from(deg_ref, NP):
    deg = deg_ref[0:NP, 0:1] + deg_ref[NP:2 * NP, 0:1] + 1.0
    return lax.rsqrt(deg)


def _k1_body(x_ref, w_ref, deg_ref, o_ref):
    NP = x_ref.shape[0]
    dinv = _dinv_from(deg_ref, NP)
    h = jnp.dot(x_ref[...], w_ref[...], preferred_element_type=jnp.float32)
    o_ref[...] = h * dinv


def _k2_body(s_ref, y_ref, deg_ref, b_ref, w_ref, o_ref):
    NP = y_ref.shape[0]
    dinv = _dinv_from(deg_ref, NP)
    s = s_ref[0:NP, :] + s_ref[NP:2 * NP, :] + y_ref[...]
    h = jnp.maximum(s * dinv + b_ref[...], 0.0)
    o_ref[...] = jnp.dot(h, w_ref[...], preferred_element_type=jnp.float32) * dinv


def _k3_body(s_ref, y_ref, deg_ref, b_ref, bt_ref, u_ref, wg_ref, bg_ref,
             wh1_ref, wh2_ref, bh_ref, o_ref):
    NP = y_ref.shape[0]
    G = u_ref.shape[0]
    dinv = _dinv_from(deg_ref, NP)
    s = s_ref[0:NP, :] + s_ref[NP:2 * NP, :] + y_ref[...]
    emb = jnp.maximum(s * dinv + b_ref[...], 0.0)
    gid = lax.broadcasted_iota(jnp.int32, (G, NP), 0)
    onehot = (gid == bt_ref[...]).astype(jnp.float32)
    sums = jnp.dot(onehot, emb, preferred_element_type=jnp.float32)
    counts = jnp.sum(onehot, axis=1, keepdims=True)
    gemb = sums / jnp.maximum(counts, 1.0)
    gl = jnp.maximum(
        jnp.dot(u_ref[...], wg_ref[...], preferred_element_type=jnp.float32)
        + bg_ref[...], 0.0)
    out = (jnp.sum(gemb * wh1_ref[...], axis=1, keepdims=True)
           + jnp.sum(gl * wh2_ref[...], axis=1, keepdims=True)
           + bh_ref[...])
    o_ref[...] = out


def kernel(x, edge_index, u, batch, batch_size, W1, b1, W2, b2, Wg, bg, Wh, bh):
    N, DIN = x.shape
    D = W1.shape[1]
    G = u.shape[0]
    E = edge_index.shape[1]
    NW = _NC * _NS
    NP = -(-(N + 1) // 256) * 256
    EC = -(-E // _CH)
    TOT = -(-EC // (NW * 8)) * (NW * 8)
    CPT = TOT // NW
    padE = TOT * _CH - E

    padE = padE + _SEG * _CH
    src = jnp.concatenate(
        [edge_index[0], jnp.zeros((padE,), edge_index.dtype)]
    ).reshape(TOT + _SEG, _CH)
    dst = jnp.concatenate(
        [edge_index[1],
         jnp.full((padE - _SEG * _CH,), N, edge_index.dtype),
         jnp.zeros((_SEG * _CH,), edge_index.dtype)]
    ).reshape(TOT + _SEG, _CH)
    x_pad = jnp.pad(x, ((0, NP - N), (0, 0)))
    batch_row = jnp.pad(batch, (0, NP - N), constant_values=G).reshape(1, NP)

    b1r = b1.reshape(1, D)
    b2r = b2.reshape(1, D)
    bgr = bg.reshape(1, D)
    wh1 = Wh[:D, 0].reshape(1, D)
    wh2 = Wh[D:, 0].reshape(1, D)
    bhr = bh.reshape(1, 1)

    CPW = TOT // _NS
    CPT1 = 80
    CPT0 = CPW - CPT1

    f32 = jnp.float32
    degp = _make_deg_kernel(NP, CPT)(dst)
    y1 = pl.pallas_call(
        _k1_body, out_shape=jax.ShapeDtypeStruct((NP, D), f32),
    )(x_pad, W1, degp)
    gs = _make_gs_kernel(NP, CPT0, CPT1, D)
    s1 = gs(y1, src, dst)
    y2 = pl.pallas_call(
        _k2_body, out_shape=jax.ShapeDtypeStruct((NP, D), f32),
    )(s1, y1, degp, b1r, W2)
    s2 = gs(y2, src, dst)
    out = pl.pallas_call(
        _k3_body, out_shape=jax.ShapeDtypeStruct((G, 1), f32),
    )(s2, y2, degp, b2r, batch_row, u, Wg, bgr, wh1, wh2, bhr)
    return out


# speedup vs baseline: 1.0292x; 1.0292x over previous
import functools

import jax
import jax.numpy as jnp
from jax import lax
from jax.experimental import pallas as pl
from jax.experimental.pallas import tpu as pltpu
from jax.experimental.pallas import tpu_sc as plsc

_NC = 2
_NS = 16
_CH = 128


def _make_deg_kernel(NP, CPT):
    DW = 16
    RPT = NP // _NS
    mesh = plsc.VectorSubcoreMesh(core_axis_name="c", subcore_axis_name="s")

    @functools.partial(
        pl.kernel,
        out_type=jax.ShapeDtypeStruct((_NC * NP, DW), jnp.float32),
        mesh=mesh,
        scratch_types=[
            pltpu.VMEM((CPT, _CH), jnp.int32),
            pltpu.VMEM((_CH, DW), jnp.float32),
            pltpu.VMEM((16, DW), jnp.float32),
            pltpu.VMEM_SHARED((NP, DW), jnp.float32),
        ],
    )
    def deg_kernel(dst_hbm, out_hbm, dst_v, ones_v, zbuf, acc):
        cid = lax.axis_index("c")
        sid = lax.axis_index("s")
        w = cid * _NS + sid
        for i in range(16):
            zbuf[i, :] = jnp.zeros((DW,), jnp.float32)
        for i in range(_CH):
            ones_v[i, :] = jnp.ones((DW,), jnp.float32)

        def zero_body(k, _):
            pltpu.sync_copy(zbuf, acc.at[pl.ds(sid * RPT + k * 16, 16)])
            return 0

        lax.fori_loop(0, RPT // 16, zero_body, 0)
        pltpu.sync_copy(dst_hbm.at[pl.ds(w * CPT, CPT)], dst_v)
        plsc.subcore_barrier()

        def body(j, _):
            pltpu.sync_copy(ones_v, acc.at[dst_v.at[j]], add=True)
            return 0

        lax.fori_loop(0, CPT, body, 0)
        plsc.subcore_barrier()
        pltpu.sync_copy(acc.at[pl.ds(sid * RPT, RPT)],
                        out_hbm.at[pl.ds(cid * NP + sid * RPT, RPT)])

    return deg_kernel


_SEG = 48


def _make_gs_kernel(NP, CPT0, CPT1, D):
    RPT = NP // _NS
    mesh = plsc.VectorSubcoreMesh(core_axis_name="c", subcore_axis_name="s")

    @functools.partial(
        pl.kernel,
        out_type=jax.ShapeDtypeStruct((_NC * NP, D), jnp.float32),
        mesh=mesh,
        scratch_types=[
            pltpu.VMEM((_SEG, _CH), jnp.int32),
            pltpu.VMEM((_SEG, _CH), jnp.int32),
            pltpu.VMEM((_CH, D), jnp.float32),
            pltpu.VMEM((_CH, D), jnp.float32),
            pltpu.VMEM((8, D), jnp.float32),
            pltpu.VMEM_SHARED((NP, D), jnp.float32),
            pltpu.SemaphoreType.DMA,
            pltpu.SemaphoreType.DMA,
        ],
    )
    def gs_kernel(y_hbm, src_hbm, dst_hbm, out_hbm,
                  src_v, dst_v, rows_a, rows_b, zbuf, acc, sem_a, sem_b):
        cid = lax.axis_index("c")
        sid = lax.axis_index("s")
        for i in range(8):
            for g in range(D // 16):
                zbuf[i, pl.ds(g * 16, 16)] = jnp.zeros((16,), jnp.float32)

        def zero_body(k, _):
            pltpu.sync_copy(zbuf, acc.at[pl.ds(sid * RPT + k * 8, 8)])
            return 0

        lax.fori_loop(0, RPT // 8, zero_body, 0)
        plsc.subcore_barrier()

        cpt = jnp.where(cid == 0, CPT0, CPT1)
        base = pl.multiple_of(
            jnp.where(cid == 0, sid * CPT0, _NS * CPT0 + sid * CPT1), 8)
        nseg = (cpt + _SEG - 1) // _SEG

        def seg_body(s, _):
            off = pl.multiple_of(base + s * _SEG, 8)
            pltpu.sync_copy(src_hbm.at[pl.ds(off, _SEG)], src_v)
            pltpu.sync_copy(dst_hbm.at[pl.ds(off, _SEG)], dst_v)
            cnt = jnp.minimum(cpt - s * _SEG, _SEG)
            pltpu.async_copy(y_hbm.at[src_v.at[0]], rows_a, sem_a)

            def pair(jj, _):
                j0 = 2 * jj
                j1 = j0 + 1
                pltpu.async_copy(y_hbm.at[src_v.at[j1]], rows_b, sem_b)
                pltpu.make_async_copy(y_hbm.at[src_v.at[j0]],
                                      rows_a, sem_a).wait()
                pltpu.sync_copy(rows_a, acc.at[dst_v.at[j0]], add=True)

                @pl.when(j0 + 2 < cnt)
                def _():
                    pltpu.async_copy(y_hbm.at[src_v.at[j0 + 2]],
                                     rows_a, sem_a)

                pltpu.make_async_copy(y_hbm.at[src_v.at[j1]],
                                      rows_b, sem_b).wait()
                pltpu.sync_copy(rows_b, acc.at[dst_v.at[j1]], add=True)
                return 0

            lax.fori_loop(0, cnt // 2, pair, 0)
            return 0

        lax.fori_loop(0, nseg, seg_body, 0)
        plsc.subcore_barrier()
        pltpu.sync_copy(acc.at[pl.ds(sid * RPT, RPT)],
                        out_hbm.at[pl.ds(cid * NP + sid * RPT, RPT)])

    return gs_kernel


def _dinv_from(deg_ref, NP):
    deg = deg_ref[0:NP, 0:1] + deg_ref[NP:2 * NP, 0:1] + 1.0
    return lax.rsqrt(deg)


def _k1_body(x_ref, w_ref, deg_ref, o_ref):
    NP = x_ref.shape[0]
    dinv = _dinv_from(deg_ref, NP)
    h = jnp.dot(x_ref[...], w_ref[...], preferred_element_type=jnp.float32)
    o_ref[...] = h * dinv


def _k2_body(s_ref, y_ref, deg_ref, b_ref, w_ref, o_ref):
    NP = y_ref.shape[0]
    dinv = _dinv_from(deg_ref, NP)
    s = s_ref[0:NP, :] + s_ref[NP:2 * NP, :] + y_ref[...]
    h = jnp.maximum(s * dinv + b_ref[...], 0.0)
    o_ref[...] = jnp.dot(h, w_ref[...], preferred_element_type=jnp.float32) * dinv


def _k3_body(s_ref, y_ref, deg_ref, b_ref, bt_ref, u_ref, wg_ref, bg_ref,
             wh1_ref, wh2_ref, bh_ref, o_ref):
    NP = y_ref.shape[0]
    G = u_ref.shape[0]
    dinv = _dinv_from(deg_ref, NP)
    s = s_ref[0:NP, :] + s_ref[NP:2 * NP, :] + y_ref[...]
    emb = jnp.maximum(s * dinv + b_ref[...], 0.0)
    gid = lax.broadcasted_iota(jnp.int32, (G, NP), 0)
    onehot = (gid == bt_ref[...]).astype(jnp.float32)
    sums = jnp.dot(onehot, emb, preferred_element_type=jnp.float32)
    counts = jnp.sum(onehot, axis=1, keepdims=True)
    gemb = sums / jnp.maximum(counts, 1.0)
    gl = jnp.maximum(
        jnp.dot(u_ref[...], wg_ref[...], preferred_element_type=jnp.float32)
        + bg_ref[...], 0.0)
    out = (jnp.sum(gemb * wh1_ref[...], axis=1, keepdims=True)
           + jnp.sum(gl * wh2_ref[...], axis=1, keepdims=True)
           + bh_ref[...])
    o_ref[...] = out


def kernel(x, edge_index, u, batch, batch_size, W1, b1, W2, b2, Wg, bg, Wh, bh):
    N, DIN = x.shape
    D = W1.shape[1]
    G = u.shape[0]
    E = edge_index.shape[1]
    NW = _NC * _NS
    NP = -(-(N + 1) // 256) * 256
    EC = -(-E // _CH)
    TOT = -(-EC // (NW * 8)) * (NW * 8)
    CPT = TOT // NW
    padE = TOT * _CH - E

    padE = padE + _SEG * _CH
    src = jnp.concatenate(
        [edge_index[0], jnp.zeros((padE,), edge_index.dtype)]
    ).reshape(TOT + _SEG, _CH)
    dst = jnp.concatenate(
        [edge_index[1],
         jnp.full((padE - _SEG * _CH,), N, edge_index.dtype),
         jnp.zeros((_SEG * _CH,), edge_index.dtype)]
    ).reshape(TOT + _SEG, _CH)
    x_pad = jnp.pad(x, ((0, NP - N), (0, 0)))
    batch_row = jnp.pad(batch, (0, NP - N), constant_values=G).reshape(1, NP)

    b1r = b1.reshape(1, D)
    b2r = b2.reshape(1, D)
    bgr = bg.reshape(1, D)
    wh1 = Wh[:D, 0].reshape(1, D)
    wh2 = Wh[D:, 0].reshape(1, D)
    bhr = bh.reshape(1, 1)

    CPW = TOT // _NS
    CPT1 = 56
    CPT0 = CPW - CPT1

    f32 = jnp.float32
    degp = _make_deg_kernel(NP, CPT)(dst)
    y1 = pl.pallas_call(
        _k1_body, out_shape=jax.ShapeDtypeStruct((NP, D), f32),
    )(x_pad, W1, degp)
    gs = _make_gs_kernel(NP, CPT0, CPT1, D)
    s1 = gs(y1, src, dst)
    y2 = pl.pallas_call(
        _k2_body, out_shape=jax.ShapeDtypeStruct((NP, D), f32),
    )(s1, y1, degp, b1r, W2)
    s2 = gs(y2, src, dst)
    out = pl.pallas_call(
        _k3_body, out_shape=jax.ShapeDtypeStruct((G, 1), f32),
    )(s2, y2, degp, b2r, batch_row, u, Wg, bgr, wh1, wh2, bhr)
    return out


# speedup vs baseline: 1.0520x; 1.0222x over previous
import functools

import jax
import jax.numpy as jnp
from jax import lax
from jax.experimental import pallas as pl
from jax.experimental.pallas import tpu as pltpu
from jax.experimental.pallas import tpu_sc as plsc

_NC = 2
_NS = 16
_CH = 128


def _make_deg_kernel(NP, CPT):
    DW = 16
    RPT = NP // _NS
    mesh = plsc.VectorSubcoreMesh(core_axis_name="c", subcore_axis_name="s")

    @functools.partial(
        pl.kernel,
        out_type=jax.ShapeDtypeStruct((_NC * NP, DW), jnp.float32),
        mesh=mesh,
        scratch_types=[
            pltpu.VMEM((CPT, _CH), jnp.int32),
            pltpu.VMEM((_CH, DW), jnp.float32),
            pltpu.VMEM((16, DW), jnp.float32),
            pltpu.VMEM_SHARED((NP, DW), jnp.float32),
        ],
    )
    def deg_kernel(dst_hbm, out_hbm, dst_v, ones_v, zbuf, acc):
        cid = lax.axis_index("c")
        sid = lax.axis_index("s")
        w = cid * _NS + sid
        for i in range(16):
            zbuf[i, :] = jnp.zeros((DW,), jnp.float32)
        for i in range(_CH):
            ones_v[i, :] = jnp.ones((DW,), jnp.float32)

        def zero_body(k, _):
            pltpu.sync_copy(zbuf, acc.at[pl.ds(sid * RPT + k * 16, 16)])
            return 0

        lax.fori_loop(0, RPT // 16, zero_body, 0)
        pltpu.sync_copy(dst_hbm.at[pl.ds(w * CPT, CPT)], dst_v)
        plsc.subcore_barrier()

        def body(j, _):
            pltpu.sync_copy(ones_v, acc.at[dst_v.at[j]], add=True)
            return 0

        lax.fori_loop(0, CPT, body, 0)
        plsc.subcore_barrier()
        pltpu.sync_copy(acc.at[pl.ds(sid * RPT, RPT)],
                        out_hbm.at[pl.ds(cid * NP + sid * RPT, RPT)])

    return deg_kernel


_SEG = 48


def _make_gs_kernel(NP, CPT0, CPT1, D):
    RPT = NP // _NS
    mesh = plsc.VectorSubcoreMesh(core_axis_name="c", subcore_axis_name="s")

    @functools.partial(
        pl.kernel,
        out_type=jax.ShapeDtypeStruct((_NC * NP, D), jnp.float32),
        mesh=mesh,
        scratch_types=[
            pltpu.VMEM((_SEG, _CH), jnp.int32),
            pltpu.VMEM((_SEG, _CH), jnp.int32),
            pltpu.VMEM((_CH, D), jnp.float32),
            pltpu.VMEM((_CH, D), jnp.float32),
            pltpu.VMEM((8, D), jnp.float32),
            pltpu.VMEM_SHARED((NP, D), jnp.float32),
            pltpu.SemaphoreType.DMA,
            pltpu.SemaphoreType.DMA,
        ],
    )
    def gs_kernel(y_hbm, src_hbm, dst_hbm, out_hbm,
                  src_v, dst_v, rows_a, rows_b, zbuf, acc, sem_a, sem_b):
        cid = lax.axis_index("c")
        sid = lax.axis_index("s")
        for i in range(8):
            for g in range(D // 16):
                zbuf[i, pl.ds(g * 16, 16)] = jnp.zeros((16,), jnp.float32)

        def zero_body(k, _):
            pltpu.sync_copy(zbuf, acc.at[pl.ds(sid * RPT + k * 8, 8)])
            return 0

        lax.fori_loop(0, RPT // 8, zero_body, 0)
        plsc.subcore_barrier()

        cpt = jnp.where(cid == 0, CPT0, CPT1)
        base = pl.multiple_of(
            jnp.where(cid == 0, sid * CPT0, _NS * CPT0 + sid * CPT1), 8)
        nseg = (cpt + _SEG - 1) // _SEG

        def seg_body(s, _):
            off = pl.multiple_of(base + s * _SEG, 8)
            pltpu.sync_copy(src_hbm.at[pl.ds(off, _SEG)], src_v)
            pltpu.sync_copy(dst_hbm.at[pl.ds(off, _SEG)], dst_v)
            cnt = jnp.minimum(cpt - s * _SEG, _SEG)
            pltpu.async_copy(y_hbm.at[src_v.at[0]], rows_a, sem_a)

            def pair(jj, _):
                j0 = 2 * jj
                j1 = j0 + 1
                pltpu.async_copy(y_hbm.at[src_v.at[j1]], rows_b, sem_b)
                pltpu.make_async_copy(y_hbm.at[src_v.at[j0]],
                                      rows_a, sem_a).wait()
                pltpu.sync_copy(rows_a, acc.at[dst_v.at[j0]], add=True)

                @pl.when(j0 + 2 < cnt)
                def _():
                    pltpu.async_copy(y_hbm.at[src_v.at[j0 + 2]],
                                     rows_a, sem_a)

                pltpu.make_async_copy(y_hbm.at[src_v.at[j1]],
                                      rows_b, sem_b).wait()
                pltpu.sync_copy(rows_b, acc.at[dst_v.at[j1]], add=True)
                return 0

            lax.fori_loop(0, cnt // 2, pair, 0)
            return 0

        lax.fori_loop(0, nseg, seg_body, 0)
        plsc.subcore_barrier()
        pltpu.sync_copy(acc.at[pl.ds(sid * RPT, RPT)],
                        out_hbm.at[pl.ds(cid * NP + sid * RPT, RPT)])

    return gs_kernel


def _dinv_from(deg_ref, NP):
    deg = deg_ref[0:NP, 0:1] + deg_ref[NP:2 * NP, 0:1] + 1.0
    return lax.rsqrt(deg)


def _k1_body(x_ref, w_ref, deg_ref, o_ref):
    NP = x_ref.shape[0]
    dinv = _dinv_from(deg_ref, NP)
    h = jnp.dot(x_ref[...], w_ref[...], preferred_element_type=jnp.float32)
    o_ref[...] = h * dinv


def _k2_body(s_ref, y_ref, deg_ref, b_ref, w_ref, o_ref):
    NP = y_ref.shape[0]
    dinv = _dinv_from(deg_ref, NP)
    s = s_ref[0:NP, :] + s_ref[NP:2 * NP, :] + y_ref[...]
    h = jnp.maximum(s * dinv + b_ref[...], 0.0)
    o_ref[...] = jnp.dot(h, w_ref[...], preferred_element_type=jnp.float32) * dinv


def _k3_body(s_ref, y_ref, deg_ref, b_ref, bt_ref, u_ref, wg_ref, bg_ref,
             wh1_ref, wh2_ref, bh_ref, o_ref):
    NP = y_ref.shape[0]
    G = u_ref.shape[0]
    dinv = _dinv_from(deg_ref, NP)
    s = s_ref[0:NP, :] + s_ref[NP:2 * NP, :] + y_ref[...]
    emb = jnp.maximum(s * dinv + b_ref[...], 0.0)
    gid = lax.broadcasted_iota(jnp.int32, (G, NP), 0)
    onehot = (gid == bt_ref[...]).astype(jnp.float32)
    sums = jnp.dot(onehot, emb, preferred_element_type=jnp.float32)
    counts = jnp.sum(onehot, axis=1, keepdims=True)
    gemb = sums / jnp.maximum(counts, 1.0)
    gl = jnp.maximum(
        jnp.dot(u_ref[...], wg_ref[...], preferred_element_type=jnp.float32)
        + bg_ref[...], 0.0)
    out = (jnp.sum(gemb * wh1_ref[...], axis=1, keepdims=True)
           + jnp.sum(gl * wh2_ref[...], axis=1, keepdims=True)
           + bh_ref[...])
    o_ref[...] = out


def kernel(x, edge_index, u, batch, batch_size, W1, b1, W2, b2, Wg, bg, Wh, bh):
    N, DIN = x.shape
    D = W1.shape[1]
    G = u.shape[0]
    E = edge_index.shape[1]
    NW = _NC * _NS
    NP = -(-(N + 1) // 256) * 256
    EC = -(-E // _CH)
    TOT = -(-EC // (NW * 8)) * (NW * 8)
    CPT = TOT // NW
    padE = TOT * _CH - E

    padE = padE + _SEG * _CH
    src = jnp.concatenate(
        [edge_index[0], jnp.zeros((padE,), edge_index.dtype)]
    ).reshape(TOT + _SEG, _CH)
    dst = jnp.concatenate(
        [edge_index[1],
         jnp.full((padE - _SEG * _CH,), N, edge_index.dtype),
         jnp.zeros((_SEG * _CH,), edge_index.dtype)]
    ).reshape(TOT + _SEG, _CH)
    x_pad = jnp.pad(x, ((0, NP - N), (0, 0)))
    batch_row = jnp.pad(batch, (0, NP - N), constant_values=G).reshape(1, NP)

    b1r = b1.reshape(1, D)
    b2r = b2.reshape(1, D)
    bgr = bg.reshape(1, D)
    wh1 = Wh[:D, 0].reshape(1, D)
    wh2 = Wh[D:, 0].reshape(1, D)
    bhr = bh.reshape(1, 1)

    CPW = TOT // _NS
    CPT1 = 24
    CPT0 = CPW - CPT1

    f32 = jnp.float32
    degp = _make_deg_kernel(NP, CPT)(dst)
    y1 = pl.pallas_call(
        _k1_body, out_shape=jax.ShapeDtypeStruct((NP, D), f32),
    )(x_pad, W1, degp)
    gs = _make_gs_kernel(NP, CPT0, CPT1, D)
    s1 = gs(y1, src, dst)
    y2 = pl.pallas_call(
        _k2_body, out_shape=jax.ShapeDtypeStruct((NP, D), f32),
    )(s1, y1, degp, b1r, W2)
    s2 = gs(y2, src, dst)
    out = pl.pallas_call(
        _k3_body, out_shape=jax.ShapeDtypeStruct((G, 1), f32),
    )(s2, y2, degp, b2r, batch_row, u, Wg, bgr, wh1, wh2, bhr)
    return out


# speedup vs baseline: 1.0931x; 1.0391x over previous
import functools

import jax
import jax.numpy as jnp
from jax import lax
from jax.experimental import pallas as pl
from jax.experimental.pallas import tpu as pltpu
from jax.experimental.pallas import tpu_sc as plsc

_NC = 2
_NS = 16
_CH = 128


def _make_deg_kernel(NP, CPT):
    DW = 16
    RPT = NP // _NS
    mesh = plsc.VectorSubcoreMesh(core_axis_name="c", subcore_axis_name="s")

    @functools.partial(
        pl.kernel,
        out_type=jax.ShapeDtypeStruct((_NC * NP, DW), jnp.float32),
        mesh=mesh,
        scratch_types=[
            pltpu.VMEM((CPT, _CH), jnp.int32),
            pltpu.VMEM((_CH, DW), jnp.float32),
            pltpu.VMEM((16, DW), jnp.float32),
            pltpu.VMEM_SHARED((NP, DW), jnp.float32),
        ],
    )
    def deg_kernel(dst_hbm, out_hbm, dst_v, ones_v, zbuf, acc):
        cid = lax.axis_index("c")
        sid = lax.axis_index("s")
        w = cid * _NS + sid
        for i in range(16):
            zbuf[i, :] = jnp.zeros((DW,), jnp.float32)
        for i in range(_CH):
            ones_v[i, :] = jnp.ones((DW,), jnp.float32)

        def zero_body(k, _):
            pltpu.sync_copy(zbuf, acc.at[pl.ds(sid * RPT + k * 16, 16)])
            return 0

        lax.fori_loop(0, RPT // 16, zero_body, 0)
        pltpu.sync_copy(dst_hbm.at[pl.ds(w * CPT, CPT)], dst_v)
        plsc.subcore_barrier()

        def body(j, _):
            pltpu.sync_copy(ones_v, acc.at[dst_v.at[j]], add=True)
            return 0

        lax.fori_loop(0, CPT, body, 0)
        plsc.subcore_barrier()
        pltpu.sync_copy(acc.at[pl.ds(sid * RPT, RPT)],
                        out_hbm.at[pl.ds(cid * NP + sid * RPT, RPT)])

    return deg_kernel


_SEG = 48


def _make_gs_kernel(NP, CPT0, CPT1, D):
    RPT = NP // _NS
    mesh = plsc.VectorSubcoreMesh(core_axis_name="c", subcore_axis_name="s")

    @functools.partial(
        pl.kernel,
        out_type=jax.ShapeDtypeStruct((_NC * NP, D), jnp.float32),
        mesh=mesh,
        scratch_types=[
            pltpu.VMEM((_SEG, _CH), jnp.int32),
            pltpu.VMEM((_SEG, _CH), jnp.int32),
            pltpu.VMEM((_CH, D), jnp.float32),
            pltpu.VMEM((_CH, D), jnp.float32),
            pltpu.VMEM((8, D), jnp.float32),
            pltpu.VMEM_SHARED((NP, D), jnp.float32),
            pltpu.SemaphoreType.DMA,
            pltpu.SemaphoreType.DMA,
        ],
    )
    def gs_kernel(y_hbm, src_hbm, dst_hbm, out_hbm,
                  src_v, dst_v, rows_a, rows_b, zbuf, acc, sem_a, sem_b):
        cid = lax.axis_index("c")
        sid = lax.axis_index("s")
        for i in range(8):
            for g in range(D // 16):
                zbuf[i, pl.ds(g * 16, 16)] = jnp.zeros((16,), jnp.float32)

        def zero_body(k, _):
            pltpu.sync_copy(zbuf, acc.at[pl.ds(sid * RPT + k * 8, 8)])
            return 0

        lax.fori_loop(0, RPT // 8, zero_body, 0)
        plsc.subcore_barrier()

        cpt = jnp.where(cid == 0, CPT0, CPT1)
        base = pl.multiple_of(
            jnp.where(cid == 0, sid * CPT0, _NS * CPT0 + sid * CPT1), 8)
        nseg = (cpt + _SEG - 1) // _SEG

        def seg_body(s, _):
            off = pl.multiple_of(base + s * _SEG, 8)
            pltpu.sync_copy(src_hbm.at[pl.ds(off, _SEG)], src_v)
            pltpu.sync_copy(dst_hbm.at[pl.ds(off, _SEG)], dst_v)
            cnt = jnp.minimum(cpt - s * _SEG, _SEG)
            pltpu.async_copy(y_hbm.at[src_v.at[0]], rows_a, sem_a)

            def pair(jj, _):
                j0 = 2 * jj
                j1 = j0 + 1
                pltpu.async_copy(y_hbm.at[src_v.at[j1]], rows_b, sem_b)
                pltpu.make_async_copy(y_hbm.at[src_v.at[j0]],
                                      rows_a, sem_a).wait()
                pltpu.sync_copy(rows_a, acc.at[dst_v.at[j0]], add=True)

                @pl.when(j0 + 2 < cnt)
                def _():
                    pltpu.async_copy(y_hbm.at[src_v.at[j0 + 2]],
                                     rows_a, sem_a)

                pltpu.make_async_copy(y_hbm.at[src_v.at[j1]],
                                      rows_b, sem_b).wait()
                pltpu.sync_copy(rows_b, acc.at[dst_v.at[j1]], add=True)
                return 0

            lax.fori_loop(0, cnt // 2, pair, 0)
            return 0

        lax.fori_loop(0, nseg, seg_body, 0)
        plsc.subcore_barrier()
        pltpu.sync_copy(acc.at[pl.ds(sid * RPT, RPT)],
                        out_hbm.at[pl.ds(cid * NP + sid * RPT, RPT)])

    return gs_kernel


def _dinv_from(deg_ref, NP):
    deg = deg_ref[0:NP, 0:1] + deg_ref[NP:2 * NP, 0:1] + 1.0
    return lax.rsqrt(deg)


def _k1_body(x_ref, w_ref, deg_ref, o_ref):
    NP = x_ref.shape[0]
    dinv = _dinv_from(deg_ref, NP)
    h = jnp.dot(x_ref[...], w_ref[...], preferred_element_type=jnp.float32)
    o_ref[...] = h * dinv


def _k2_body(s_ref, y_ref, deg_ref, b_ref, w_ref, o_ref):
    NP = y_ref.shape[0]
    dinv = _dinv_from(deg_ref, NP)
    s = s_ref[0:NP, :] + s_ref[NP:2 * NP, :] + y_ref[...]
    h = jnp.maximum(s * dinv + b_ref[...], 0.0)
    o_ref[...] = jnp.dot(h, w_ref[...], preferred_element_type=jnp.float32) * dinv


def _k3_body(s_ref, y_ref, deg_ref, b_ref, bt_ref, u_ref, wg_ref, bg_ref,
             wh1_ref, wh2_ref, bh_ref, o_ref):
    NP = y_ref.shape[0]
    G = u_ref.shape[0]
    dinv = _dinv_from(deg_ref, NP)
    s = s_ref[0:NP, :] + s_ref[NP:2 * NP, :] + y_ref[...]
    emb = jnp.maximum(s * dinv + b_ref[...], 0.0)
    gid = lax.broadcasted_iota(jnp.int32, (G, NP), 0)
    onehot = (gid == bt_ref[...]).astype(jnp.float32)
    sums = jnp.dot(onehot, emb, preferred_element_type=jnp.float32)
    counts = jnp.sum(onehot, axis=1, keepdims=True)
    gemb = sums / jnp.maximum(counts, 1.0)
    gl = jnp.maximum(
        jnp.dot(u_ref[...], wg_ref[...], preferred_element_type=jnp.float32)
        + bg_ref[...], 0.0)
    out = (jnp.sum(gemb * wh1_ref[...], axis=1, keepdims=True)
           + jnp.sum(gl * wh2_ref[...], axis=1, keepdims=True)
           + bh_ref[...])
    o_ref[...] = out


def kernel(x, edge_index, u, batch, batch_size, W1, b1, W2, b2, Wg, bg, Wh, bh):
    N, DIN = x.shape
    D = W1.shape[1]
    G = u.shape[0]
    E = edge_index.shape[1]
    NW = _NC * _NS
    NP = -(-(N + 1) // 256) * 256
    EC = -(-E // _CH)
    TOT = -(-EC // (NW * 8)) * (NW * 8)
    CPT = TOT // NW
    padE = TOT * _CH - E

    padE = padE + _SEG * _CH
    src = jnp.concatenate(
        [edge_index[0], jnp.zeros((padE,), edge_index.dtype)]
    ).reshape(TOT + _SEG, _CH)
    dst = jnp.concatenate(
        [edge_index[1],
         jnp.full((padE - _SEG * _CH,), N, edge_index.dtype),
         jnp.zeros((_SEG * _CH,), edge_index.dtype)]
    ).reshape(TOT + _SEG, _CH)
    x_pad = jnp.pad(x, ((0, NP - N), (0, 0)))
    batch_row = jnp.pad(batch, (0, NP - N), constant_values=G).reshape(1, NP)

    b1r = b1.reshape(1, D)
    b2r = b2.reshape(1, D)
    bgr = bg.reshape(1, D)
    wh1 = Wh[:D, 0].reshape(1, D)
    wh2 = Wh[D:, 0].reshape(1, D)
    bhr = bh.reshape(1, 1)

    CPW = TOT // _NS
    CPT1 = 8
    CPT0 = CPW - CPT1

    f32 = jnp.float32
    degp = _make_deg_kernel(NP, CPT)(dst)
    y1 = pl.pallas_call(
        _k1_body, out_shape=jax.ShapeDtypeStruct((NP, D), f32),
    )(x_pad, W1, degp)
    gs = _make_gs_kernel(NP, CPT0, CPT1, D)
    s1 = gs(y1, src, dst)
    y2 = pl.pallas_call(
        _k2_body, out_shape=jax.ShapeDtypeStruct((NP, D), f32),
    )(s1, y1, degp, b1r, W2)
    s2 = gs(y2, src, dst)
    out = pl.pallas_call(
        _k3_body, out_shape=jax.ShapeDtypeStruct((G, 1), f32),
    )(s2, y2, degp, b2r, batch_row, u, Wg, bgr, wh1, wh2, bhr)
    return out
